# baseline (speedup 1.0000x reference)
import jax
import jax.numpy as jnp
from jax.experimental import pallas as pl
from jax.experimental.pallas import tpu as pltpu

_BLOCK_M = 2048
_BLOCK_K = 512


def _fused_head_kernel(x_ref, w_ref, b_ref, o_ref, acc_ref):
    k = pl.program_id(1)
    nk = pl.num_programs(1)
    part = jnp.dot(x_ref[...], w_ref[...], preferred_element_type=jnp.float32)

    @pl.when(k == 0)
    def _():
        acc_ref[...] = part

    @pl.when(k > 0)
    def _():
        acc_ref[...] += part

    @pl.when(k == nk - 1)
    def _():
        o_ref[...] = jax.nn.sigmoid(acc_ref[...] + b_ref[...])


def kernel(x, W_heads, b_heads, W_free, b_free):
    tokens, d = x.shape
    n_comp, _, comp_size = W_heads.shape
    n_out = n_comp * comp_size + W_free.shape[1]
    W_all = jnp.concatenate(
        [jnp.transpose(W_heads, (1, 0, 2)).reshape(d, n_comp * comp_size), W_free],
        axis=1,
    )
    b_all = jnp.concatenate([b_heads.reshape(-1), b_free])[None, :]

    bm = min(_BLOCK_M, tokens)
    bk = min(_BLOCK_K, d)
    return pl.pallas_call(
        _fused_head_kernel,
        grid=(pl.cdiv(tokens, bm), pl.cdiv(d, bk)),
        in_specs=[
            pl.BlockSpec((bm, bk), lambda i, k: (i, k)),
            pl.BlockSpec((bk, n_out), lambda i, k: (k, 0)),
            pl.BlockSpec((1, n_out), lambda i, k: (0, 0)),
        ],
        out_specs=pl.BlockSpec((bm, n_out), lambda i, k: (i, 0)),
        out_shape=jax.ShapeDtypeStruct((tokens, n_out), x.dtype),
        scratch_shapes=[pltpu.VMEM((bm, n_out), jnp.float32)],
        compiler_params=pltpu.CompilerParams(
            dimension_semantics=("parallel", "arbitrary"),
        ),
    )(x, W_all, b_all)


# speedup vs baseline: 1.1576x; 1.1576x over previous
import jax
import jax.numpy as jnp
from jax.experimental import pallas as pl
from jax.experimental.pallas import tpu as pltpu

_BLOCK_M = 2048


def _fused_head_kernel(x_ref, w_ref, b_ref, o_ref):
    acc = jnp.dot(x_ref[...], w_ref[...], preferred_element_type=jnp.float32)
    o_ref[...] = jax.nn.sigmoid(acc + b_ref[...])


def kernel(x, W_heads, b_heads, W_free, b_free):
    tokens, d = x.shape
    n_comp, _, comp_size = W_heads.shape
    n_out = n_comp * comp_size + W_free.shape[1]
    W_all = jnp.concatenate(
        [jnp.transpose(W_heads, (1, 0, 2)).reshape(d, n_comp * comp_size), W_free],
        axis=1,
    )
    b_all = jnp.concatenate([b_heads.reshape(-1), b_free])[None, :]

    bm = min(_BLOCK_M, tokens)
    return pl.pallas_call(
        _fused_head_kernel,
        grid=(pl.cdiv(tokens, bm),),
        in_specs=[
            pl.BlockSpec((bm, d), lambda i: (i, 0)),
            pl.BlockSpec((d, n_out), lambda i: (0, 0)),
            pl.BlockSpec((1, n_out), lambda i: (0, 0)),
        ],
        out_specs=pl.BlockSpec((bm, n_out), lambda i: (i, 0)),
        out_shape=jax.ShapeDtypeStruct((tokens, n_out), x.dtype),
        compiler_params=pltpu.CompilerParams(
            dimension_semantics=("arbitrary",),
        ),
    )(x, W_all, b_all)
